# Initial kernel scaffold; baseline (speedup 1.0000x reference)
import jax
import jax.numpy as jnp
from jax.experimental import pallas as pl


def kernel(AssetName, ShapeIndex, Owner, Pips, ControlGroup, Cloak, Continuous, unit_name_table, unit_shape_table, owner_table, pip_table, control_table, cloak_table):
    raise NotImplementedError("write your pallas kernel here")


# speedup vs baseline: 7.8700x; 7.8700x over previous
import jax
import jax.numpy as jnp
from jax import lax
from jax.experimental import pallas as pl
from jax.experimental.pallas import tpu as pltpu
from jax.experimental.pallas import tpu_sc as plsc

MAX_SHAPES = 8
N_PIPS = 5
OUT_D = 44
NC, NS, L = 2, 16, 16
NW = NC * NS
CHUNK = 128


def _body(asset_h, shape_h, owner_h, pips_h, ctrl_h, cloak_h, cont_h,
          name_t_h, shape_t_h, owner_t_h, pip_t_h, ctrl_t_h, cloak_t_h,
          out_h,
          asset2d, shp2d, sidx2d, owner_v, ctrl_v, cloak_v, pips_v, cont_v,
          name_rows, shape_rows, out_v, owner_t, pip_t, ctrl_t, cloak_t,
          sem):
  rpw = name_rows.shape[0]
  nch = rpw // CHUNK
  wid = lax.axis_index("s") * NC + lax.axis_index("c")
  base = wid * rpw

  for j in range(nch):
    pltpu.sync_copy(asset_h.at[pl.ds(base + j * CHUNK, CHUNK)], asset2d.at[j])
    pltpu.sync_copy(shape_h.at[pl.ds(base + j * CHUNK, CHUNK)], shp2d.at[j])
  for j in range(nch):
    for m in range(CHUNK // L):
      a = asset2d[j, pl.ds(m * L, L)]
      s = shp2d[j, pl.ds(m * L, L)]
      sidx2d[j, pl.ds(m * L, L)] = a * MAX_SHAPES + s

  copies = []
  for j in range(nch):
    copies.append(pltpu.async_copy(
        name_t_h.at[asset2d.at[j]], name_rows.at[pl.ds(j * CHUNK, CHUNK)], sem))
    copies.append(pltpu.async_copy(
        shape_t_h.at[sidx2d.at[j]], shape_rows.at[pl.ds(j * CHUNK, CHUNK)], sem))

  pltpu.sync_copy(owner_h.at[pl.ds(base, rpw)], owner_v)
  pltpu.sync_copy(ctrl_h.at[pl.ds(base, rpw)], ctrl_v)
  pltpu.sync_copy(cloak_h.at[pl.ds(base, rpw)], cloak_v)
  pltpu.sync_copy(pips_h.at[pl.ds(base * N_PIPS, rpw * N_PIPS)], pips_v)
  pltpu.sync_copy(cont_h.at[pl.ds(base * N_PIPS, rpw * N_PIPS)], cont_v)
  pltpu.sync_copy(owner_t_h, owner_t)
  pltpu.sync_copy(pip_t_h, pip_t)
  pltpu.sync_copy(ctrl_t_h, ctrl_t)
  pltpu.sync_copy(cloak_t_h, cloak_t)

  iota16 = lax.iota(jnp.int32, L)

  def small_block(b, carry):
    r0 = b * L
    rows = r0 + iota16
    rows_o = rows * OUT_D
    rows_p = rows * N_PIPS
    ov = owner_v[pl.ds(r0, L)] * 3
    for c in range(3):
      v = plsc.load_gather(owner_t, [ov + c])
      plsc.store_scatter(out_v, [rows_o + (16 + c)], v)
    for p in range(N_PIPS):
      pv = plsc.load_gather(pips_v, [rows_p + p]) * 3
      for c in range(3):
        v = plsc.load_gather(pip_t, [pv + c])
        plsc.store_scatter(out_v, [rows_o + (19 + 3 * p + c)], v)
    cv = ctrl_v[pl.ds(r0, L)] * 3
    for c in range(3):
      v = plsc.load_gather(ctrl_t, [cv + c])
      plsc.store_scatter(out_v, [rows_o + (34 + c)], v)
    kv = cloak_v[pl.ds(r0, L)] * 2
    for c in range(2):
      v = plsc.load_gather(cloak_t, [kv + c])
      plsc.store_scatter(out_v, [rows_o + (37 + c)], v)
    for c in range(N_PIPS):
      v = plsc.load_gather(cont_v, [rows_p + c])
      plsc.store_scatter(out_v, [rows_o + (39 + c)], v)
    return carry

  lax.fori_loop(0, rpw // L, small_block, 0)

  for cp in copies:
    cp.wait()

  def unit_block(r, carry):
    u = name_rows[r, :] + shape_rows[r, :]
    plsc.store_scatter(out_v, [r * OUT_D + iota16], u)
    return carry

  lax.fori_loop(0, rpw, unit_block, 0)

  pltpu.sync_copy(out_v, out_h.at[pl.ds(base * OUT_D, rpw * OUT_D)])


def kernel(AssetName, ShapeIndex, Owner, Pips, ControlGroup, Cloak, Continuous,
           unit_name_table, unit_shape_table, owner_table, pip_table,
           control_table, cloak_table):
  b = AssetName.shape[0]
  rpw = b // NW
  nch = rpw // CHUNK
  i32 = jnp.int32
  f32 = jnp.float32
  run = pl.kernel(
      _body,
      out_type=jax.ShapeDtypeStruct((b * OUT_D,), f32),
      mesh=plsc.VectorSubcoreMesh(core_axis_name="c", subcore_axis_name="s"),
      compiler_params=pltpu.CompilerParams(needs_layout_passes=False,
                                           use_tc_tiling_on_sc=False),
      scratch_types=[
          pltpu.VMEM((nch, CHUNK), i32),
          pltpu.VMEM((nch, CHUNK), i32),
          pltpu.VMEM((nch, CHUNK), i32),
          pltpu.VMEM((rpw,), i32),
          pltpu.VMEM((rpw,), i32),
          pltpu.VMEM((rpw,), i32),
          pltpu.VMEM((rpw * N_PIPS,), i32),
          pltpu.VMEM((rpw * N_PIPS,), f32),
          pltpu.VMEM((rpw, 16), f32),
          pltpu.VMEM((rpw, 16), f32),
          pltpu.VMEM((rpw * OUT_D,), f32),
          pltpu.VMEM((owner_table.size,), f32),
          pltpu.VMEM((pip_table.size,), f32),
          pltpu.VMEM((control_table.size,), f32),
          pltpu.VMEM((cloak_table.size,), f32),
          pltpu.SemaphoreType.DMA,
      ],
  )
  out = run(AssetName.astype(i32), ShapeIndex.astype(i32),
            Owner.astype(i32), Pips.astype(i32).reshape(-1),
            ControlGroup.astype(i32), Cloak.astype(i32),
            Continuous.reshape(-1),
            unit_name_table, unit_shape_table,
            owner_table.reshape(-1), pip_table.reshape(-1),
            control_table.reshape(-1), cloak_table.reshape(-1))
  return out.reshape(b, OUT_D)


# speedup vs baseline: 8.3008x; 1.0547x over previous
import jax
import jax.numpy as jnp
from jax import lax
from jax.experimental import pallas as pl
from jax.experimental.pallas import tpu as pltpu
from jax.experimental.pallas import tpu_sc as plsc

MAX_SHAPES = 8
N_PIPS = 5
OUT_D = 44
NC, NS, L = 2, 16, 16
NW = NC * NS
CHUNK = 128


def _body(asset_h, shape_h, owner_h, pips_h, ctrl_h, cloak_h, cont_h,
          name_t_h, shape_t_h, owner_t_h, pip_t_h, ctrl_t_h, cloak_t_h,
          out_h,
          asset2d, shp2d, sidx2d, owner_v, ctrl_v, cloak_v, pips_v, cont_v,
          name_rows, shape_rows, out_v, owner_t, pip_t, ctrl_t, cloak_t,
          sem_i, sem_m, sem_u):
  rpw = name_rows.shape[0]
  nch = rpw // CHUNK
  wid = lax.axis_index("s") * NC + lax.axis_index("c")
  base = wid * rpw

  idx_cps = []
  for j in range(nch):
    idx_cps.append(pltpu.async_copy(
        asset_h.at[pl.ds(base + j * CHUNK, CHUNK)], asset2d.at[j], sem_i))
    idx_cps.append(pltpu.async_copy(
        shape_h.at[pl.ds(base + j * CHUNK, CHUNK)], shp2d.at[j], sem_i))
  misc_cps = [
      pltpu.async_copy(owner_h.at[pl.ds(base, rpw)], owner_v, sem_m),
      pltpu.async_copy(ctrl_h.at[pl.ds(base, rpw)], ctrl_v, sem_m),
      pltpu.async_copy(cloak_h.at[pl.ds(base, rpw)], cloak_v, sem_m),
      pltpu.async_copy(pips_h.at[pl.ds(base * N_PIPS, rpw * N_PIPS)],
                       pips_v, sem_m),
      pltpu.async_copy(cont_h.at[pl.ds(base * N_PIPS, rpw * N_PIPS)],
                       cont_v, sem_m),
      pltpu.async_copy(owner_t_h, owner_t, sem_m),
      pltpu.async_copy(pip_t_h, pip_t, sem_m),
      pltpu.async_copy(ctrl_t_h, ctrl_t, sem_m),
      pltpu.async_copy(cloak_t_h, cloak_t, sem_m),
  ]
  for cp in idx_cps:
    cp.wait()

  for j in range(nch):
    for m in range(CHUNK // L):
      a = asset2d[j, pl.ds(m * L, L)]
      s = shp2d[j, pl.ds(m * L, L)]
      sidx2d[j, pl.ds(m * L, L)] = a * MAX_SHAPES + s
  unit_cps = []
  for j in range(nch):
    unit_cps.append(pltpu.async_copy(
        name_t_h.at[asset2d.at[j]], name_rows.at[pl.ds(j * CHUNK, CHUNK)],
        sem_u))
    unit_cps.append(pltpu.async_copy(
        shape_t_h.at[sidx2d.at[j]], shape_rows.at[pl.ds(j * CHUNK, CHUNK)],
        sem_u))
  for cp in misc_cps:
    cp.wait()
  for cp in unit_cps:
    cp.wait()

  iota16 = lax.iota(jnp.int32, L)
  iota_o = iota16 * OUT_D
  iota_p = iota16 * N_PIPS
  unit_cols = [c * OUT_D + iota16 for c in range(L)]

  @plsc.parallel_loop(0, rpw // L, unroll=2)
  def blk(b):
    r0 = b * L
    o0 = r0 * OUT_D
    rows_o = o0 + iota_o
    rows_p = r0 * N_PIPS + iota_p
    for i in range(L):
      u = name_rows[r0 + i, :] + shape_rows[r0 + i, :]
      plsc.store_scatter(out_v, [o0 + unit_cols[i]], u)
    ov = owner_v[pl.ds(r0, L)] * 3
    for c in range(3):
      v = plsc.load_gather(owner_t, [ov + c])
      plsc.store_scatter(out_v, [rows_o + (16 + c)], v)
    for p in range(N_PIPS):
      pv = plsc.load_gather(pips_v, [rows_p + p]) * 3
      for c in range(3):
        v = plsc.load_gather(pip_t, [pv + c])
        plsc.store_scatter(out_v, [rows_o + (19 + 3 * p + c)], v)
    cv = ctrl_v[pl.ds(r0, L)] * 3
    for c in range(3):
      v = plsc.load_gather(ctrl_t, [cv + c])
      plsc.store_scatter(out_v, [rows_o + (34 + c)], v)
    kv = cloak_v[pl.ds(r0, L)] * 2
    for c in range(2):
      v = plsc.load_gather(cloak_t, [kv + c])
      plsc.store_scatter(out_v, [rows_o + (37 + c)], v)
    for c in range(N_PIPS):
      v = plsc.load_gather(cont_v, [rows_p + c])
      plsc.store_scatter(out_v, [rows_o + (39 + c)], v)

  pltpu.sync_copy(out_v, out_h.at[pl.ds(base * OUT_D, rpw * OUT_D)])


def kernel(AssetName, ShapeIndex, Owner, Pips, ControlGroup, Cloak, Continuous,
           unit_name_table, unit_shape_table, owner_table, pip_table,
           control_table, cloak_table):
  b = AssetName.shape[0]
  rpw = b // NW
  nch = rpw // CHUNK
  i32 = jnp.int32
  f32 = jnp.float32
  run = pl.kernel(
      _body,
      out_type=jax.ShapeDtypeStruct((b * OUT_D,), f32),
      mesh=plsc.VectorSubcoreMesh(core_axis_name="c", subcore_axis_name="s"),
      compiler_params=pltpu.CompilerParams(needs_layout_passes=False,
                                           use_tc_tiling_on_sc=False),
      scratch_types=[
          pltpu.VMEM((nch, CHUNK), i32),
          pltpu.VMEM((nch, CHUNK), i32),
          pltpu.VMEM((nch, CHUNK), i32),
          pltpu.VMEM((rpw,), i32),
          pltpu.VMEM((rpw,), i32),
          pltpu.VMEM((rpw,), i32),
          pltpu.VMEM((rpw * N_PIPS,), i32),
          pltpu.VMEM((rpw * N_PIPS,), f32),
          pltpu.VMEM((rpw, 16), f32),
          pltpu.VMEM((rpw, 16), f32),
          pltpu.VMEM((rpw * OUT_D,), f32),
          pltpu.VMEM((owner_table.size,), f32),
          pltpu.VMEM((pip_table.size,), f32),
          pltpu.VMEM((control_table.size,), f32),
          pltpu.VMEM((cloak_table.size,), f32),
          pltpu.SemaphoreType.DMA,
          pltpu.SemaphoreType.DMA,
          pltpu.SemaphoreType.DMA,
      ],
  )
  out = run(AssetName.astype(i32), ShapeIndex.astype(i32),
            Owner.astype(i32), Pips.astype(i32).reshape(-1),
            ControlGroup.astype(i32), Cloak.astype(i32),
            Continuous.reshape(-1),
            unit_name_table, unit_shape_table,
            owner_table.reshape(-1), pip_table.reshape(-1),
            control_table.reshape(-1), cloak_table.reshape(-1))
  return out.reshape(b, OUT_D)


# speedup vs baseline: 16.3607x; 1.9710x over previous
import jax
import jax.numpy as jnp
from jax import lax
from jax.experimental import pallas as pl
from jax.experimental.pallas import tpu as pltpu
from jax.experimental.pallas import tpu_sc as plsc

MAX_SHAPES = 8
N_PIPS = 5
OUT_D = 44
NC, NS, L = 2, 16, 16
NW = NC * NS
CHUNK = 128


def _body(asset_h, shape_h, owner_h, pips_h, ctrl_h, cloak_h, cont_h,
          name_t_h, shape_t_h, owner_t_h, pip_t_h, ctrl_t_h, cloak_t_h,
          out_h,
          asset2d, shp2d, sidx2d, owner_v, ctrl_v, cloak_v, pips_c, cont_c,
          name_rows, shape_rows, out_v, owner_t, pip_t, ctrl_t, cloak_t,
          sem_i, sem_m, sem_u, sem_o):
  rpw = name_rows.shape[0]
  nch = rpw // CHUNK
  wid = lax.axis_index("s") * NC + lax.axis_index("c")
  base = wid * rpw

  idx_cps = []
  for j in range(nch):
    idx_cps.append(pltpu.async_copy(
        asset_h.at[pl.ds(base + j * CHUNK, CHUNK)], asset2d.at[j], sem_i))
    idx_cps.append(pltpu.async_copy(
        shape_h.at[pl.ds(base + j * CHUNK, CHUNK)], shp2d.at[j], sem_i))
  misc_cps = [
      pltpu.async_copy(owner_h.at[pl.ds(base, rpw)], owner_v, sem_m),
      pltpu.async_copy(ctrl_h.at[pl.ds(base, rpw)], ctrl_v, sem_m),
      pltpu.async_copy(cloak_h.at[pl.ds(base, rpw)], cloak_v, sem_m),
      pltpu.async_copy(owner_t_h, owner_t, sem_m),
      pltpu.async_copy(pip_t_h, pip_t, sem_m),
      pltpu.async_copy(ctrl_t_h, ctrl_t, sem_m),
      pltpu.async_copy(cloak_t_h, cloak_t, sem_m),
  ]
  for p in range(N_PIPS):
    misc_cps.append(pltpu.async_copy(
        pips_h.at[p, pl.ds(base, rpw)], pips_c.at[p], sem_m))
    misc_cps.append(pltpu.async_copy(
        cont_h.at[p, pl.ds(base, rpw)], cont_c.at[p], sem_m))
  for cp in idx_cps:
    cp.wait()

  for j in range(nch):
    for m in range(CHUNK // L):
      a = asset2d[j, pl.ds(m * L, L)]
      s = shp2d[j, pl.ds(m * L, L)]
      sidx2d[j, pl.ds(m * L, L)] = a * MAX_SHAPES + s
  unit_cps = []
  for j in range(nch):
    unit_cps.append(pltpu.async_copy(
        name_t_h.at[asset2d.at[j]], name_rows.at[pl.ds(j * CHUNK, CHUNK)],
        sem_u))
    unit_cps.append(pltpu.async_copy(
        shape_t_h.at[sidx2d.at[j]], shape_rows.at[pl.ds(j * CHUNK, CHUNK)],
        sem_u))
  for cp in misc_cps:
    cp.wait()
  for cp in unit_cps:
    cp.wait()

  iota16 = lax.iota(jnp.int32, L)
  col_base = iota16 * rpw

  @plsc.parallel_loop(0, rpw // L, unroll=2)
  def blk(b):
    r0 = b * L
    for i in range(L):
      u = name_rows[r0 + i, :] + shape_rows[r0 + i, :]
      plsc.store_scatter(out_v, [col_base + (r0 + i)], u)
    ov = owner_v[pl.ds(r0, L)]
    for c in range(3):
      v = plsc.load_gather(owner_t, [ov + c * 256])
      out_v[pl.ds((16 + c) * rpw + r0, L)] = v
    for p in range(N_PIPS):
      pv = pips_c[p, pl.ds(r0, L)]
      for c in range(3):
        v = plsc.load_gather(pip_t, [pv + c * 10])
        out_v[pl.ds((19 + 3 * p + c) * rpw + r0, L)] = v
    cv = ctrl_v[pl.ds(r0, L)]
    for c in range(3):
      v = plsc.load_gather(ctrl_t, [cv + c * 256])
      out_v[pl.ds((34 + c) * rpw + r0, L)] = v
    kv = cloak_v[pl.ds(r0, L)]
    for c in range(2):
      v = plsc.load_gather(cloak_t, [kv + c * 5])
      out_v[pl.ds((37 + c) * rpw + r0, L)] = v
    for c in range(N_PIPS):
      out_v[pl.ds((39 + c) * rpw + r0, L)] = cont_c[c, pl.ds(r0, L)]

  out_cps = [pltpu.async_copy(out_v.at[pl.ds(c * rpw, rpw)],
                              out_h.at[c, pl.ds(base, rpw)], sem_o)
             for c in range(OUT_D)]
  for cp in out_cps:
    cp.wait()


def kernel(AssetName, ShapeIndex, Owner, Pips, ControlGroup, Cloak, Continuous,
           unit_name_table, unit_shape_table, owner_table, pip_table,
           control_table, cloak_table):
  b = AssetName.shape[0]
  rpw = b // NW
  nch = rpw // CHUNK
  i32 = jnp.int32
  f32 = jnp.float32
  run = pl.kernel(
      _body,
      out_type=jax.ShapeDtypeStruct((OUT_D, b), f32),
      mesh=plsc.VectorSubcoreMesh(core_axis_name="c", subcore_axis_name="s"),
      compiler_params=pltpu.CompilerParams(needs_layout_passes=False,
                                           use_tc_tiling_on_sc=False),
      scratch_types=[
          pltpu.VMEM((nch, CHUNK), i32),
          pltpu.VMEM((nch, CHUNK), i32),
          pltpu.VMEM((nch, CHUNK), i32),
          pltpu.VMEM((rpw,), i32),
          pltpu.VMEM((rpw,), i32),
          pltpu.VMEM((rpw,), i32),
          pltpu.VMEM((N_PIPS, rpw), i32),
          pltpu.VMEM((N_PIPS, rpw), f32),
          pltpu.VMEM((rpw, 16), f32),
          pltpu.VMEM((rpw, 16), f32),
          pltpu.VMEM((rpw * OUT_D,), f32),
          pltpu.VMEM((owner_table.size,), f32),
          pltpu.VMEM((pip_table.size,), f32),
          pltpu.VMEM((control_table.size,), f32),
          pltpu.VMEM((cloak_table.size,), f32),
          pltpu.SemaphoreType.DMA,
          pltpu.SemaphoreType.DMA,
          pltpu.SemaphoreType.DMA,
          pltpu.SemaphoreType.DMA,
      ],
  )
  out = run(AssetName.astype(i32), ShapeIndex.astype(i32),
            Owner.astype(i32), Pips.astype(i32).T, ControlGroup.astype(i32),
            Cloak.astype(i32), Continuous.T,
            unit_name_table, unit_shape_table,
            owner_table.T.reshape(-1), pip_table.T.reshape(-1),
            control_table.T.reshape(-1), cloak_table.T.reshape(-1))
  return out.T


# speedup vs baseline: 17.4031x; 1.0637x over previous
import jax
import jax.numpy as jnp
from jax import lax
from jax.experimental import pallas as pl
from jax.experimental.pallas import tpu as pltpu
from jax.experimental.pallas import tpu_sc as plsc

MAX_SHAPES = 8
N_PIPS = 5
OUT_D = 44
NC, NS, L = 2, 16, 16
NW = NC * NS
CHUNK = 128
OWNER_OFF, PIP_OFF, CTRL_OFF, CLOAK_OFF = 0, 768, 798, 1566
TABLES_LEN = 1576


def _body(asset_h, shape_h, owner_h, pips_h, ctrl_h, cloak_h, cont_h,
          name_t_h, shape_t_h, tables_h,
          out_h,
          asset2d, shp2d, sidx2d, owner_v, ctrl_v, cloak_v, pips_c, cont_c,
          name_rows, shape_rows, out_v, tables_t,
          sem_i, sem_m, sem_u, sem_o):
  rpw = name_rows.shape[0]
  nch = rpw // CHUNK
  wid = lax.axis_index("s") * NC + lax.axis_index("c")
  base = wid * rpw

  idx_cps = []
  for j in range(nch):
    idx_cps.append(pltpu.async_copy(
        asset_h.at[pl.ds(base + j * CHUNK, CHUNK)], asset2d.at[j], sem_i))
    idx_cps.append(pltpu.async_copy(
        shape_h.at[pl.ds(base + j * CHUNK, CHUNK)], shp2d.at[j], sem_i))
  misc_cps = [
      pltpu.async_copy(owner_h.at[pl.ds(base, rpw)], owner_v, sem_m),
      pltpu.async_copy(ctrl_h.at[pl.ds(base, rpw)], ctrl_v, sem_m),
      pltpu.async_copy(cloak_h.at[pl.ds(base, rpw)], cloak_v, sem_m),
      pltpu.async_copy(tables_h, tables_t, sem_m),
  ]
  for p in range(N_PIPS):
    misc_cps.append(pltpu.async_copy(
        pips_h.at[p, pl.ds(base, rpw)], pips_c.at[p], sem_m))
    misc_cps.append(pltpu.async_copy(
        cont_h.at[p, pl.ds(base, rpw)], cont_c.at[p], sem_m))
  for cp in idx_cps:
    cp.wait()

  for j in range(nch):
    for m in range(CHUNK // L):
      a = asset2d[j, pl.ds(m * L, L)]
      s = shp2d[j, pl.ds(m * L, L)]
      sidx2d[j, pl.ds(m * L, L)] = a * MAX_SHAPES + s
  unit_cps = []
  for j in range(nch):
    unit_cps.append(pltpu.async_copy(
        name_t_h.at[asset2d.at[j]], name_rows.at[pl.ds(j * CHUNK, CHUNK)],
        sem_u))
    unit_cps.append(pltpu.async_copy(
        shape_t_h.at[sidx2d.at[j]], shape_rows.at[pl.ds(j * CHUNK, CHUNK)],
        sem_u))
  for cp in misc_cps:
    cp.wait()

  @plsc.parallel_loop(0, rpw // L, unroll=2)
  def small_blk(b):
    r0 = b * L
    ov = owner_v[pl.ds(r0, L)]
    for c in range(3):
      v = plsc.load_gather(tables_t, [ov + (OWNER_OFF + c * 256)])
      out_v[pl.ds((16 + c) * rpw + r0, L)] = v
    for p in range(N_PIPS):
      pv = pips_c[p, pl.ds(r0, L)]
      for c in range(3):
        v = plsc.load_gather(tables_t, [pv + (PIP_OFF + c * 10)])
        out_v[pl.ds((19 + 3 * p + c) * rpw + r0, L)] = v
    cv = ctrl_v[pl.ds(r0, L)]
    for c in range(3):
      v = plsc.load_gather(tables_t, [cv + (CTRL_OFF + c * 256)])
      out_v[pl.ds((34 + c) * rpw + r0, L)] = v
    kv = cloak_v[pl.ds(r0, L)]
    for c in range(2):
      v = plsc.load_gather(tables_t, [kv + (CLOAK_OFF + c * 5)])
      out_v[pl.ds((37 + c) * rpw + r0, L)] = v
    for c in range(N_PIPS):
      out_v[pl.ds((39 + c) * rpw + r0, L)] = cont_c[c, pl.ds(r0, L)]

  for cp in unit_cps:
    cp.wait()

  iota16 = lax.iota(jnp.int32, L)
  col_base = iota16 * rpw

  @plsc.parallel_loop(0, rpw, unroll=4)
  def unit_blk(r):
    u = name_rows[r, :] + shape_rows[r, :]
    plsc.store_scatter(out_v, [col_base + r], u)

  out_cps = [pltpu.async_copy(out_v.at[pl.ds(c * rpw, rpw)],
                              out_h.at[c, pl.ds(base, rpw)], sem_o)
             for c in range(OUT_D)]
  for cp in out_cps:
    cp.wait()


def kernel(AssetName, ShapeIndex, Owner, Pips, ControlGroup, Cloak, Continuous,
           unit_name_table, unit_shape_table, owner_table, pip_table,
           control_table, cloak_table):
  b = AssetName.shape[0]
  rpw = b // NW
  nch = rpw // CHUNK
  i32 = jnp.int32
  f32 = jnp.float32
  tables = jnp.concatenate([
      owner_table.T.reshape(-1), pip_table.T.reshape(-1),
      control_table.T.reshape(-1), cloak_table.T.reshape(-1)])
  run = pl.kernel(
      _body,
      out_type=jax.ShapeDtypeStruct((OUT_D, b), f32),
      mesh=plsc.VectorSubcoreMesh(core_axis_name="c", subcore_axis_name="s"),
      compiler_params=pltpu.CompilerParams(needs_layout_passes=False,
                                           use_tc_tiling_on_sc=False),
      scratch_types=[
          pltpu.VMEM((nch, CHUNK), i32),
          pltpu.VMEM((nch, CHUNK), i32),
          pltpu.VMEM((nch, CHUNK), i32),
          pltpu.VMEM((rpw,), i32),
          pltpu.VMEM((rpw,), i32),
          pltpu.VMEM((rpw,), i32),
          pltpu.VMEM((N_PIPS, rpw), i32),
          pltpu.VMEM((N_PIPS, rpw), f32),
          pltpu.VMEM((rpw, 16), f32),
          pltpu.VMEM((rpw, 16), f32),
          pltpu.VMEM((rpw * OUT_D,), f32),
          pltpu.VMEM((TABLES_LEN,), f32),
          pltpu.SemaphoreType.DMA,
          pltpu.SemaphoreType.DMA,
          pltpu.SemaphoreType.DMA,
          pltpu.SemaphoreType.DMA,
      ],
  )
  out = run(AssetName.astype(i32), ShapeIndex.astype(i32),
            Owner.astype(i32), Pips.astype(i32).T, ControlGroup.astype(i32),
            Cloak.astype(i32), Continuous.T,
            unit_name_table, unit_shape_table, tables)
  return out.T
